# Initial kernel scaffold; baseline (speedup 1.0000x reference)
import jax
import jax.numpy as jnp
from jax.experimental import pallas as pl


def kernel(tokens, table):
    raise NotImplementedError("write your pallas kernel here")


# speedup vs baseline: 2.1746x; 2.1746x over previous
import functools
import math

import jax
import jax.numpy as jnp
from jax import lax
from jax.experimental import pallas as pl
from jax.experimental.pallas import tpu as pltpu
from jax.experimental.pallas import tpu_sc as plsc

VOCAB = 100000
EMB = 128
SCALE = math.sqrt(float(EMB))

NC = 2
NS = 16
NW = NC * NS
B = 4096 * 50
BPW = B // NW
CHUNK = 128
NCHUNK = BPW // CHUNK
ROW_BLK = VOCAB // 125


def _scale_body(t_ref, o_ref):
    o_ref[...] = t_ref[...] * SCALE


def _prescale(table):
    return pl.pallas_call(
        _scale_body,
        out_shape=jax.ShapeDtypeStruct((VOCAB, EMB), jnp.float32),
        grid=(VOCAB // ROW_BLK,),
        in_specs=[pl.BlockSpec((ROW_BLK, EMB), lambda i: (i, 0))],
        out_specs=pl.BlockSpec((ROW_BLK, EMB), lambda i: (i, 0)),
    )(table)


@functools.partial(
    pl.kernel,
    out_type=jax.ShapeDtypeStruct((B, EMB), jnp.float32),
    mesh=plsc.VectorSubcoreMesh(core_axis_name="c", subcore_axis_name="s"),
    scratch_types=[
        pltpu.VMEM((NCHUNK, CHUNK), jnp.int32),
        pltpu.VMEM((CHUNK, EMB), jnp.float32),
        pltpu.SemaphoreType.DMA,
    ],
)
def _sc_gather(idx_hbm, table_hbm, out_hbm, idx_v, rows_v, sem):
    wid = lax.axis_index("s") * NC + lax.axis_index("c")
    pltpu.sync_copy(idx_hbm.at[wid], idx_v)

    def body(g, carry):
        pltpu.async_copy(table_hbm.at[idx_v.at[g]], rows_v, sem).wait()
        pltpu.sync_copy(rows_v, out_hbm.at[pl.ds(wid * BPW + g * CHUNK, CHUNK)])
        return carry

    lax.fori_loop(0, NCHUNK, body, 0)


def kernel(tokens, table):
    idx = tokens.reshape(NW, NCHUNK, CHUNK).astype(jnp.int32)
    scaled = _prescale(table)
    out = _sc_gather(idx, scaled)
    return out.reshape(tokens.shape[0], tokens.shape[1], EMB)


# speedup vs baseline: 3.0070x; 1.3828x over previous
import functools
import math

import jax
import jax.numpy as jnp
from jax import lax
from jax.experimental import pallas as pl
from jax.experimental.pallas import tpu as pltpu
from jax.experimental.pallas import tpu_sc as plsc

VOCAB = 100000
EMB = 128
SCALE = math.sqrt(float(EMB))

NC = 2
NS = 16
NW = NC * NS
B = 4096 * 50
BPW = B // NW
CHUNK = 128
NCHUNK = BPW // CHUNK
ROW_BLK = VOCAB // 125


def _scale_body(t_ref, o_ref):
    o_ref[...] = t_ref[...] * SCALE


def _prescale(table):
    return pl.pallas_call(
        _scale_body,
        out_shape=jax.ShapeDtypeStruct((VOCAB, EMB), jnp.float32),
        grid=(VOCAB // ROW_BLK,),
        in_specs=[pl.BlockSpec((ROW_BLK, EMB), lambda i: (i, 0))],
        out_specs=pl.BlockSpec((ROW_BLK, EMB), lambda i: (i, 0)),
    )(table)


NSEQ = 4096
SEQ = 50
SPW = NSEQ // NW
SEQ_PER_CHUNK = 2
CHUNK_ROWS = SEQ_PER_CHUNK * SEQ
NCHUNK2 = SPW // SEQ_PER_CHUNK


@functools.partial(
    pl.kernel,
    out_type=jax.ShapeDtypeStruct((NSEQ, SEQ, EMB), jnp.float32),
    mesh=plsc.VectorSubcoreMesh(core_axis_name="c", subcore_axis_name="s"),
    scratch_types=[
        pltpu.VMEM((NCHUNK2, CHUNK_ROWS), jnp.int32),
        pltpu.VMEM((CHUNK_ROWS, EMB), jnp.float32),
        pltpu.SemaphoreType.DMA,
    ],
)
def _sc_gather(idx_hbm, table_hbm, out_hbm, idx_v, rows_v, sem):
    wid = lax.axis_index("s") * NC + lax.axis_index("c")
    pltpu.sync_copy(idx_hbm.at[wid], idx_v)

    def body(g, carry):
        pltpu.async_copy(table_hbm.at[idx_v.at[g]], rows_v, sem).wait()
        b0 = wid * SPW + g * SEQ_PER_CHUNK
        pltpu.sync_copy(rows_v.at[pl.ds(0, SEQ)], out_hbm.at[b0])
        pltpu.sync_copy(rows_v.at[pl.ds(SEQ, SEQ)], out_hbm.at[b0 + 1])
        return carry

    lax.fori_loop(0, NCHUNK2, body, 0)


def kernel(tokens, table):
    idx = tokens.reshape(NW, NCHUNK2, CHUNK_ROWS).astype(jnp.int32)
    scaled = _prescale(table)
    return _sc_gather(idx, scaled)


# speedup vs baseline: 5.0744x; 1.6876x over previous
import functools
import math

import jax
import jax.numpy as jnp
from jax import lax
from jax.experimental import pallas as pl
from jax.experimental.pallas import tpu as pltpu
from jax.experimental.pallas import tpu_sc as plsc

VOCAB = 100000
EMB = 128
SCALE = math.sqrt(float(EMB))

NC = 2
NS = 16
NW = NC * NS
LANES = 16

NSEQ = 4096
SEQ = 50
SPW = NSEQ // NW
SEQ_PER_CHUNK = 2
CHUNK_ROWS = SEQ_PER_CHUNK * SEQ
NCHUNK = SPW // SEQ_PER_CHUNK
EMB_VREGS = EMB // LANES


@functools.partial(
    pl.kernel,
    out_type=jax.ShapeDtypeStruct((NSEQ, SEQ, EMB), jnp.float32),
    mesh=plsc.VectorSubcoreMesh(core_axis_name="c", subcore_axis_name="s"),
    scratch_types=[
        pltpu.VMEM((NCHUNK, CHUNK_ROWS), jnp.int32),
        pltpu.VMEM((CHUNK_ROWS, EMB), jnp.float32),
        pltpu.VMEM((CHUNK_ROWS, EMB), jnp.float32),
        pltpu.SemaphoreType.DMA,
        pltpu.SemaphoreType.DMA,
    ],
)
def _sc_gather(idx_hbm, table_hbm, out_hbm, idx_v, rows0, rows1, sem0, sem1):
    wid = lax.axis_index("s") * NC + lax.axis_index("c")
    pltpu.sync_copy(idx_hbm.at[wid], idx_v)

    def start_gather(g, buf, sem):
        pltpu.async_copy(table_hbm.at[idx_v.at[g]], buf, sem)

    def wait_gather(g, buf, sem):
        pltpu.make_async_copy(table_hbm.at[idx_v.at[g]], buf, sem).wait()

    def scale_rows(buf):
        def row_body(r, carry):
            for c in range(EMB_VREGS):
                sl = pl.ds(c * LANES, LANES)
                buf[r, sl] = buf[r, sl] * SCALE
            return carry

        lax.fori_loop(0, CHUNK_ROWS, row_body, 0)

    def write_out(g, buf):
        b0 = wid * SPW + g * SEQ_PER_CHUNK
        pltpu.sync_copy(buf.at[pl.ds(0, SEQ)], out_hbm.at[b0])
        pltpu.sync_copy(buf.at[pl.ds(SEQ, SEQ)], out_hbm.at[b0 + 1])

    start_gather(0, rows0, sem0)

    def body(h, carry):
        start_gather(2 * h + 1, rows1, sem1)
        wait_gather(2 * h, rows0, sem0)
        scale_rows(rows0)
        write_out(2 * h, rows0)

        @pl.when(h + 1 < NCHUNK // 2)
        def _():
            start_gather(2 * h + 2, rows0, sem0)

        wait_gather(2 * h + 1, rows1, sem1)
        scale_rows(rows1)
        write_out(2 * h + 1, rows1)
        return carry

    lax.fori_loop(0, NCHUNK // 2, body, 0)


def kernel(tokens, table):
    idx = tokens.reshape(NW, NCHUNK, CHUNK_ROWS).astype(jnp.int32)
    return _sc_gather(idx, table)
